# Initial kernel scaffold; baseline (speedup 1.0000x reference)
import jax
import jax.numpy as jnp
from jax.experimental import pallas as pl


def kernel(x, edge_index, W0, attn_l0, attn_r0, b0, W1, attn_l1, attn_r1, res_W1, b1):
    raise NotImplementedError("write your pallas kernel here")


# speedup vs baseline: 9.4486x; 9.4486x over previous
import functools

import jax
import jax.numpy as jnp
from jax import lax
from jax.experimental import pallas as pl
from jax.experimental.pallas import tpu as pltpu
from jax.experimental.pallas import tpu_sc as plsc

_F32 = jnp.float32
_I32 = jnp.int32
_EPS = 1e-9
_C0 = 40
_C1 = 128


def _tc1(x, wt, wr, wd, bn):
    n = x.shape[0]
    nb = n // bn

    def body(x_ref, wt_ref, wr_ref, wd_ref, ts_ref, erb_ref, td_ref):
        xb = x_ref[...]
        ts_ref[...] = jnp.dot(xb, wt_ref[0], preferred_element_type=_F32)
        erb_ref[...] = jnp.dot(xb, wr_ref[0], preferred_element_type=_F32)
        td = jnp.dot(xb, wd_ref[0], preferred_element_type=_F32)
        td_ref[...] = jnp.concatenate(
            [td, jnp.zeros((td.shape[0], 112), _F32)], axis=1)

    return pl.pallas_call(
        body,
        grid=(nb, 2),
        in_specs=[
            pl.BlockSpec((bn, 128), lambda i, c: (i, 0)),
            pl.BlockSpec((1, 128, 256), lambda i, c: (c, 0, 0)),
            pl.BlockSpec((1, 128, 128), lambda i, c: (c, 0, 0)),
            pl.BlockSpec((1, 128, 16), lambda i, c: (c, 0, 0)),
        ],
        out_specs=[
            pl.BlockSpec((bn, 256), lambda i, c: (c * nb + i, 0)),
            pl.BlockSpec((bn, 128), lambda i, c: (c * nb + i, 0)),
            pl.BlockSpec((bn, 128), lambda i, c: (c * nb + i, 0)),
        ],
        out_shape=[
            jax.ShapeDtypeStruct((2 * n, 256), _F32),
            jax.ShapeDtypeStruct((2 * n, 128), _F32),
            jax.ShapeDtypeStruct((2 * n, 128), _F32),
        ],
    )(x, wt, wr, wd)


def _tc2(num0, den0, b0r, wcat, bn):
    n2 = num0.shape[0]
    n = n2 // 2
    nb = n // bn

    def body(na_ref, nb_ref, da_ref, db_ref, b0_ref, wc_ref,
             t1_ref, rs_ref, er_ref):
        den = da_ref[...] + db_ref[...]

        def half(nblk, h0):
            cols = []
            for h in range(4):
                seg = nblk[:, 32 * h:32 * h + 32] / (
                    den[:, h0 + h:h0 + h + 1] + _EPS)
                cols.append(seg)
            return jnp.concatenate(cols, axis=1)

        ha = half(na_ref[...], 0)
        hb = half(nb_ref[...], 4)
        h1 = jnp.maximum(
            jnp.concatenate([ha, hb], axis=1) + b0_ref[...], 0.0)
        p = jnp.dot(h1, wc_ref[...], preferred_element_type=_F32)
        z64 = jnp.zeros((p.shape[0], 64), _F32)
        t1_ref[...] = jnp.concatenate([p[:, 0:64], z64], axis=1)
        rs_ref[...] = p[:, 64:96]
        er_ref[...] = jnp.concatenate(
            [p[:, 96:128], z64, jnp.zeros((p.shape[0], 32), _F32)], axis=1)

    return pl.pallas_call(
        body,
        grid=(nb,),
        in_specs=[
            pl.BlockSpec((bn, 128), lambda i: (i, 0)),
            pl.BlockSpec((bn, 128), lambda i: (i + nb, 0)),
            pl.BlockSpec((bn, 128), lambda i: (i, 0)),
            pl.BlockSpec((bn, 128), lambda i: (i + nb, 0)),
            pl.BlockSpec((1, 256), lambda i: (0, 0)),
            pl.BlockSpec((256, 128), lambda i: (0, 0)),
        ],
        out_specs=[
            pl.BlockSpec((bn, 128), lambda i: (i, 0)),
            pl.BlockSpec((bn, 32), lambda i: (i, 0)),
            pl.BlockSpec((bn, 128), lambda i: (i, 0)),
        ],
        out_shape=[
            jax.ShapeDtypeStruct((n, 128), _F32),
            jax.ShapeDtypeStruct((n, 32), _F32),
            jax.ShapeDtypeStruct((n, 128), _F32),
        ],
    )(num0, num0, den0, den0, b0r, wcat)


def _tc3(num1, res1, b1r, bn):
    n = res1.shape[0]
    nb = n // bn

    def body(na_ref, nb_ref, rs_ref, b1_ref, o_ref):
        tot = na_ref[...] + nb_ref[...]
        o_ref[...] = (tot[:, 0:32] / (tot[:, 32:64] + _EPS)
                      + rs_ref[...] + b1_ref[...])

    return pl.pallas_call(
        body,
        grid=(nb,),
        in_specs=[
            pl.BlockSpec((bn, 128), lambda i: (i, 0)),
            pl.BlockSpec((bn, 128), lambda i: (i + nb, 0)),
            pl.BlockSpec((bn, 32), lambda i: (i, 0)),
            pl.BlockSpec((1, 32), lambda i: (0, 0)),
        ],
        out_specs=pl.BlockSpec((bn, 32), lambda i: (i, 0)),
        out_shape=jax.ShapeDtypeStruct((n, 32), _F32),
    )(num1, num1, res1, b1r)


def _sc_edge_l0(src, dst, src2, dst2, tsrc, erbt, z128, n_nodes):
    e_total = src.shape[0]
    per_tile = e_total // 16
    n_full = per_tile // _C0
    mesh = plsc.VectorSubcoreMesh(core_axis_name="c", subcore_axis_name="s")

    @functools.partial(
        pl.kernel,
        out_type=jax.ShapeDtypeStruct((2 * n_nodes, 128), _F32),
        mesh=mesh,
        scratch_types=[
            pltpu.VMEM((_C0,), _I32),
            pltpu.VMEM((_C0,), _I32),
            pltpu.VMEM((_C0,), _I32),
            pltpu.VMEM((_C0, 256), _F32),
            pltpu.VMEM((_C0, 128), _F32),
            pltpu.VMEM((_C0, 128), _F32),
            pltpu.VMEM_SHARED((n_nodes, 128), _F32),
            pltpu.SemaphoreType.DMA,
        ],
    )
    def k(src_h, dst_h, src2_h, dst2_h, ts_h, erb_h, z128_h,
          num_h,
          srco, dsto, dstr, tsbuf, erbuf, msgbuf,
          acc_sh, sem):
        cid = lax.axis_index("c")
        sid = lax.axis_index("s")
        coff = cid * e_total

        @pl.when(sid == 0)
        def _():
            pltpu.sync_copy(z128_h, acc_sh)
        plsc.subcore_barrier()

        ebase = sid * per_tile

        def do_chunk(base):
            pltpu.sync_copy(src2_h.at[pl.ds(coff + base, _C0)], srco)
            pltpu.sync_copy(dst2_h.at[pl.ds(coff + base, _C0)], dsto)
            pltpu.sync_copy(dst_h.at[pl.ds(base, _C0)], dstr)
            pltpu.async_copy(ts_h.at[srco], tsbuf, sem).wait()
            pltpu.async_copy(erb_h.at[dsto], erbuf, sem).wait()

            def edge(e, _):
                for j in range(8):
                    sl = pl.ds(j * 16, 16)
                    z = tsbuf[e, pl.ds(128 + j * 16, 16)] + erbuf[e, sl]
                    w = jnp.exp(jnp.maximum(z, 0.2 * z))
                    msgbuf[e, sl] = tsbuf[e, sl] * w
                return 0
            lax.fori_loop(0, _C0, edge, 0)
            pltpu.sync_copy(msgbuf, acc_sh.at[dstr], add=True)

        def chunk(kk, _):
            do_chunk(ebase + kk * _C0)
            return 0
        lax.fori_loop(0, n_full, chunk, 0)

        plsc.subcore_barrier()

        @pl.when(sid == 0)
        def _():
            pltpu.sync_copy(acc_sh, num_h.at[pl.ds(cid * n_nodes, n_nodes)])

    return k(src, dst, src2, dst2, tsrc, erbt, z128)


def _sc_den_l0(src, dst, dst2, td, z128, n_nodes):
    e_total = src.shape[0]
    per_core = e_total // 2
    per_tile = per_core // 16
    n_full = per_tile // _C0
    mesh = plsc.VectorSubcoreMesh(core_axis_name="c", subcore_axis_name="s")

    @functools.partial(
        pl.kernel,
        out_type=jax.ShapeDtypeStruct((2 * n_nodes, 128), _F32),
        mesh=mesh,
        scratch_types=[
            pltpu.VMEM((_C0,), _I32),
            pltpu.VMEM((_C0,), _I32),
            pltpu.VMEM((_C0,), _I32),
            pltpu.VMEM((_C0, 128), _F32),
            pltpu.VMEM((_C0, 128), _F32),
            pltpu.VMEM((_C0, 128), _F32),
            pltpu.VMEM_SHARED((n_nodes, 128), _F32),
            pltpu.SemaphoreType.DMA,
        ],
    )
    def k(src_h, dst_h, dst2_h, td_h, z128_h,
          den_h,
          srcr, dstn, dstr, elc, erc, wc,
          acc_sh, sem):
        cid = lax.axis_index("c")
        sid = lax.axis_index("s")

        @pl.when(sid == 0)
        def _():
            pltpu.sync_copy(z128_h, acc_sh)

        def zrow(e, _):
            for j in range(1, 8):
                wc[e, pl.ds(j * 16, 16)] = jnp.zeros((16,), _F32)
            return 0
        lax.fori_loop(0, _C0, zrow, 0)
        plsc.subcore_barrier()

        ebase = cid * per_core + sid * per_tile

        def do_chunk(base):
            pltpu.sync_copy(src_h.at[pl.ds(base, _C0)], srcr)
            pltpu.sync_copy(dst2_h.at[pl.ds(e_total + base, _C0)], dstn)
            pltpu.sync_copy(dst_h.at[pl.ds(base, _C0)], dstr)
            pltpu.async_copy(td_h.at[srcr], elc, sem).wait()
            pltpu.async_copy(td_h.at[dstn], erc, sem).wait()

            def dedge(e, _):
                zd = elc[e, pl.ds(0, 16)] + erc[e, pl.ds(0, 16)]
                wc[e, pl.ds(0, 16)] = jnp.exp(jnp.maximum(zd, 0.2 * zd))
                return 0
            lax.fori_loop(0, _C0, dedge, 0)
            pltpu.sync_copy(wc, acc_sh.at[dstr], add=True)

        def chunk(kk, _):
            do_chunk(ebase + kk * _C0)
            return 0
        lax.fori_loop(0, n_full, chunk, 0)

        plsc.subcore_barrier()

        @pl.when(sid == 0)
        def _():
            pltpu.sync_copy(acc_sh, den_h.at[pl.ds(cid * n_nodes, n_nodes)])

    return k(src, dst, dst2, td, z128)


def _sc_edge_l1(src, dst, t1, er1, z128, n_nodes):
    e_total = src.shape[0]
    per_core = e_total // 2
    per_tile = per_core // 16
    n_full = per_tile // _C1
    tail = per_tile - n_full * _C1
    mesh = plsc.VectorSubcoreMesh(core_axis_name="c", subcore_axis_name="s")

    @functools.partial(
        pl.kernel,
        out_type=jax.ShapeDtypeStruct((2 * n_nodes, 128), _F32),
        mesh=mesh,
        scratch_types=[
            pltpu.VMEM((_C1,), _I32),
            pltpu.VMEM((_C1,), _I32),
            pltpu.VMEM((tail,), _I32),
            pltpu.VMEM((tail,), _I32),
            pltpu.VMEM((_C1, 128), _F32),
            pltpu.VMEM((_C1, 128), _F32),
            pltpu.VMEM((_C1, 128), _F32),
            pltpu.VMEM_SHARED((n_nodes, 128), _F32),
            pltpu.SemaphoreType.DMA,
        ],
    )
    def k(src_h, dst_h, t1_h, er1_h, z128_h, num_h,
          srcv, dstv, srcv_t, dstv_t, t1buf, er1buf, msgbuf, acc_sh, sem):
        cid = lax.axis_index("c")
        sid = lax.axis_index("s")

        @pl.when(sid == 0)
        def _():
            pltpu.sync_copy(z128_h, acc_sh)

        def zrow(e, _):
            for j in range(4, 8):
                msgbuf[e, pl.ds(j * 16, 16)] = jnp.zeros((16,), _F32)
            return 0
        lax.fori_loop(0, _C1, zrow, 0)
        plsc.subcore_barrier()

        ebase = cid * per_core + sid * per_tile

        def do_chunk(base, sv, dv, cc):
            pltpu.sync_copy(src_h.at[pl.ds(base, cc)], sv)
            pltpu.sync_copy(dst_h.at[pl.ds(base, cc)], dv)
            pltpu.async_copy(t1_h.at[sv], t1buf.at[pl.ds(0, cc)], sem).wait()
            pltpu.async_copy(er1_h.at[dv], er1buf.at[pl.ds(0, cc)],
                             sem).wait()

            def edge(e, _):
                for j in range(2):
                    sl = pl.ds(j * 16, 16)
                    z = t1buf[e, pl.ds(32 + j * 16, 16)] + er1buf[e, sl]
                    w = jnp.exp(jnp.maximum(z, 0.2 * z))
                    msgbuf[e, sl] = t1buf[e, sl] * w
                    msgbuf[e, pl.ds(32 + j * 16, 16)] = w
                return 0
            lax.fori_loop(0, cc, edge, 0)
            pltpu.sync_copy(msgbuf.at[pl.ds(0, cc)], acc_sh.at[dv], add=True)

        def chunk(kk, _):
            do_chunk(ebase + kk * _C1, srcv, dstv, _C1)
            return 0
        lax.fori_loop(0, n_full, chunk, 0)
        if tail:
            do_chunk(ebase + n_full * _C1, srcv_t, dstv_t, tail)

        plsc.subcore_barrier()

        @pl.when(sid == 0)
        def _():
            pltpu.sync_copy(acc_sh, num_h.at[pl.ds(cid * n_nodes, n_nodes)])

    return k(src, dst, t1, er1, z128)


def kernel(x, edge_index, W0, attn_l0, attn_r0, b0,
           W1, attn_l1, attn_r1, res_W1, b1):
    n = x.shape[0]
    src = edge_index[0]
    dst = edge_index[1]

    w0r = W0.reshape(128, 8, 32)
    alw8 = jnp.einsum("khj,hj->kh", w0r, attn_l0)
    arw8 = jnp.einsum("khj,hj->kh", w0r, attn_r0)
    alw_exp = jnp.repeat(alw8, 32, axis=1)
    arw_exp = jnp.repeat(arw8, 32, axis=1)
    z8 = jnp.zeros((128, 8), _F32)
    wt = jnp.stack([
        jnp.concatenate([W0[:, 0:128], alw_exp[:, 0:128]], axis=1),
        jnp.concatenate([W0[:, 128:256], alw_exp[:, 128:256]], axis=1),
    ])
    wr = jnp.stack([arw_exp[:, 0:128], arw_exp[:, 128:256]])
    wd = jnp.stack([
        jnp.concatenate([alw8, z8], axis=1),
        jnp.concatenate([arw8, z8], axis=1),
    ])

    w1r = W1.reshape(256, 1, 32)
    a1l = jnp.einsum("khj,hj->kh", w1r, attn_l1)
    a1r = jnp.einsum("khj,hj->kh", w1r, attn_r1)
    wcat = jnp.concatenate(
        [W1, jnp.tile(a1l, (1, 32)), res_W1, jnp.tile(a1r, (1, 32))],
        axis=1)
    b0r = b0.reshape(1, 256)
    b1r = b1.reshape(1, 32)

    src2 = jnp.concatenate([src, src + n])
    dst2 = jnp.concatenate([dst, dst + n])
    z128 = jnp.zeros((n, 128), _F32)

    tsrc, erbt, td = _tc1(x, wt, wr, wd, bn=1000)
    num0 = _sc_edge_l0(src, dst, src2, dst2, tsrc, erbt, z128, n)
    den0 = _sc_den_l0(src, dst, dst2, td, z128, n)
    t1, res1, er1 = _tc2(num0, den0, b0r, wcat, bn=1000)
    num1 = _sc_edge_l1(src, dst, t1, er1, z128, n)
    out = _tc3(num1, res1, b1r, bn=1000)
    return out
